# baseline (speedup 1.0000x reference)
import functools

import jax
import jax.numpy as jnp
from jax import lax
from jax.experimental import pallas as pl
from jax.experimental.pallas import tpu as pltpu
from jax.experimental.pallas import tpu_sc as plsc

_N = 10000
_D = 128
_E = 320000

_NP = 10240
_NW = 32
_C = 128
_K = 80
_EW = _K * _C
_EPAD = _NW * _EW
_RPT = _NP // 16
_NBUF = 2
_PH = 2
_KH = _K // _PH

_BN = 1024
_GRID = _NP // _BN


def _make_sc_aggregate(with_deg):
    mesh = plsc.VectorSubcoreMesh(core_axis_name="c", subcore_axis_name="s")

    out_type = [jax.ShapeDtypeStruct((2, _NP, _D), jnp.float32)]
    scratch = [
        pltpu.VMEM((_KH, _C), jnp.int32),
        pltpu.VMEM((_KH, _C), jnp.int32),
        pltpu.VMEM_SHARED((_NP, _D), jnp.float32),
    ] + [pltpu.VMEM((_C, _D), jnp.float32) for _ in range(_NBUF)] \
      + [pltpu.SemaphoreType.DMA for _ in range(_NBUF)]
    if with_deg:
        out_type.append(jax.ShapeDtypeStruct((2, _NP), jnp.float32))
        scratch += [
            pltpu.VMEM((_C,), jnp.float32),
            pltpu.VMEM_SHARED((_NP,), jnp.float32),
        ]

    @functools.partial(pl.kernel, out_type=tuple(out_type), mesh=mesh,
                       scratch_types=tuple(scratch))
    def agg(h_hbm, src_hbm, dst_hbm, z2d_hbm, z1d_hbm, *rest):
        if with_deg:
            (agg_out, deg_out, src_v, dst_v, sh_agg,
             b0, b1, s0, s1, ones_v, sh_deg) = rest
        else:
            (agg_out, src_v, dst_v, sh_agg, b0, b1, s0, s1) = rest
        rows = (b0, b1)
        sems = (s0, s1)

        cid = lax.axis_index("c")
        sid = lax.axis_index("s")
        wid = cid * 16 + sid

        r0 = sid * _RPT
        pltpu.sync_copy(z2d_hbm, sh_agg.at[pl.ds(r0, _RPT), :])
        if with_deg:
            pltpu.sync_copy(z1d_hbm, sh_deg.at[pl.ds(r0, _RPT)])
            for i in range(_C // 16):
                ones_v[pl.ds(i * 16, 16)] = jnp.full((16,), 1.0, jnp.float32)

        plsc.subcore_barrier()

        for phase in range(_PH):
            pltpu.sync_copy(src_hbm.at[wid, pl.ds(phase * _KH, _KH), :], src_v)
            pltpu.sync_copy(dst_hbm.at[wid, pl.ds(phase * _KH, _KH), :], dst_v)

            for b in range(_NBUF):
                pltpu.async_copy(h_hbm.at[src_v.at[b]], rows[b], sems[b])

            def body(i, carry):
                for b in range(_NBUF):
                    j = i * _NBUF + b
                    pltpu.make_async_copy(h_hbm.at[src_v.at[j]], rows[b],
                                          sems[b]).wait()
                    pltpu.sync_copy(rows[b], sh_agg.at[dst_v.at[j]], add=True)
                    if with_deg:
                        pltpu.sync_copy(ones_v, sh_deg.at[dst_v.at[j]],
                                        add=True)
                    jn = jnp.minimum(j + _NBUF, _KH - 1)
                    pltpu.async_copy(h_hbm.at[src_v.at[jn]], rows[b], sems[b])
                return carry

            lax.fori_loop(0, _KH // _NBUF, body, 0)

            for b in range(_NBUF):
                pltpu.make_async_copy(h_hbm.at[src_v.at[_KH - 1]], rows[b],
                                      sems[b]).wait()

        plsc.subcore_barrier()

        pltpu.sync_copy(sh_agg.at[pl.ds(r0, _RPT), :],
                        agg_out.at[cid, pl.ds(r0, _RPT), :])

        if with_deg:
            @pl.when(sid == 0)
            def _():
                pltpu.sync_copy(sh_deg, deg_out.at[cid])

    return agg


def _make_tc_layer(relu):
    def body(h_ref, agg_ref, deg_ref, ws_ref, wn_ref, b_ref, o_ref):
        d = deg_ref[0] + deg_ref[1]
        inv = 1.0 / jnp.maximum(d, 1.0)
        hn = (agg_ref[0] + agg_ref[1]) * inv
        acc = jnp.dot(h_ref[...], ws_ref[...], preferred_element_type=jnp.float32)
        acc = acc + jnp.dot(hn, wn_ref[...], preferred_element_type=jnp.float32)
        acc = acc + b_ref[...]
        if relu:
            acc = jnp.maximum(acc, 0.0)
        o_ref[...] = acc

    return pl.pallas_call(
        body,
        grid=(_GRID,),
        in_specs=[
            pl.BlockSpec((_BN, _D), lambda i: (i, 0)),
            pl.BlockSpec((2, _BN, _D), lambda i: (0, i, 0)),
            pl.BlockSpec((2, _BN, 1), lambda i: (0, i, 0)),
            pl.BlockSpec((_D, _D), lambda i: (0, 0)),
            pl.BlockSpec((_D, _D), lambda i: (0, 0)),
            pl.BlockSpec((1, _D), lambda i: (0, 0)),
        ],
        out_specs=pl.BlockSpec((_BN, _D), lambda i: (i, 0)),
        out_shape=jax.ShapeDtypeStruct((_NP, _D), jnp.float32),
    )


def kernel(x, edge_index, W_self0, W_neigh0, b0, W_self1, W_neigh1, b1):
    src = edge_index[0]
    dst = edge_index[1]
    extra = _EPAD - _E
    src3 = jnp.concatenate([src, jnp.zeros((extra,), jnp.int32)]).reshape(_NW, _K, _C)
    dst3 = jnp.concatenate([dst, jnp.full((extra,), _NP - 8, jnp.int32)]).reshape(_NW, _K, _C)
    x_pad = jnp.pad(x, ((0, _NP - _N), (0, 0)))
    z2d = jnp.zeros((_RPT, _D), jnp.float32)
    z1d = jnp.zeros((_RPT,), jnp.float32)
    b0r = b0.reshape(1, _D)
    b1r = b1.reshape(1, _D)

    agg0, deg = _make_sc_aggregate(True)(x_pad, src3, dst3, z2d, z1d)
    deg3 = deg.reshape(2, _NP, 1)
    h1 = _make_tc_layer(True)(x_pad, agg0, deg3, W_self0, W_neigh0, b0r)
    (agg1,) = _make_sc_aggregate(False)(h1, src3, dst3, z2d, z1d)
    out = _make_tc_layer(False)(h1, agg1, deg3, W_self1, W_neigh1, b1r)
    return out[:_N]


# speedup vs baseline: 3.0564x; 3.0564x over previous
import functools

import jax
import jax.numpy as jnp
from jax import lax
from jax.experimental import pallas as pl
from jax.experimental.pallas import tpu as pltpu
from jax.experimental.pallas import tpu_sc as plsc

_N = 10000
_D = 128
_E = 320000

_NP = 10240
_NW = 32
_C = 128
_K = 80
_EW = _K * _C
_EPAD = _NW * _EW
_RPT = _NP // 16
_NBUF = 2
_PH = 2
_KH = _K // _PH

_BN = 1024
_GRID = _NP // _BN


def _make_sc_aggregate(with_deg):
    mesh = plsc.VectorSubcoreMesh(core_axis_name="c", subcore_axis_name="s")

    out_type = [jax.ShapeDtypeStruct((2, _NP, _D), jnp.float32)]
    scratch = [
        pltpu.VMEM((_KH, _C), jnp.int32),
        pltpu.VMEM((_KH, _C), jnp.int32),
        pltpu.VMEM_SHARED((_NP, _D), jnp.float32),
    ] + [pltpu.VMEM((_C, _D), jnp.float32) for _ in range(_NBUF)] \
      + [pltpu.SemaphoreType.DMA for _ in range(_NBUF)]
    if with_deg:
        out_type.append(jax.ShapeDtypeStruct((2, _NP), jnp.float32))
        scratch += [
            pltpu.VMEM((_C,), jnp.float32),
            pltpu.VMEM_SHARED((_NP,), jnp.float32),
        ]

    @functools.partial(pl.kernel, out_type=tuple(out_type), mesh=mesh,
                       scratch_types=tuple(scratch))
    def agg(h_hbm, src_hbm, dst_hbm, z2d_hbm, z1d_hbm, *rest):
        if with_deg:
            (agg_out, deg_out, src_v, dst_v, sh_agg,
             b0, b1, s0, s1, ones_v, sh_deg) = rest
        else:
            (agg_out, src_v, dst_v, sh_agg, b0, b1, s0, s1) = rest
        rows = (b0, b1)
        sems = (s0, s1)

        cid = lax.axis_index("c")
        sid = lax.axis_index("s")
        wid = cid * 16 + sid

        r0 = sid * _RPT
        pltpu.sync_copy(z2d_hbm, sh_agg.at[pl.ds(r0, _RPT), :])
        if with_deg:
            pltpu.sync_copy(z1d_hbm, sh_deg.at[pl.ds(r0, _RPT)])
            for i in range(_C // 16):
                ones_v[pl.ds(i * 16, 16)] = jnp.full((16,), 1.0, jnp.float32)

        plsc.subcore_barrier()

        for phase in range(_PH):
            pltpu.sync_copy(src_hbm.at[wid, pl.ds(phase * _KH, _KH), :], src_v)
            pltpu.sync_copy(dst_hbm.at[wid, pl.ds(phase * _KH, _KH), :], dst_v)

            for b in range(_NBUF):
                pltpu.async_copy(h_hbm.at[src_v.at[b]], rows[b], sems[b])

            def body(i, carry):
                for b in range(_NBUF):
                    j = i * _NBUF + b
                    pltpu.make_async_copy(h_hbm.at[src_v.at[j]], rows[b],
                                          sems[b]).wait()
                    pltpu.sync_copy(rows[b], sh_agg.at[dst_v.at[j]], add=True)
                    if with_deg:
                        pltpu.sync_copy(ones_v, sh_deg.at[dst_v.at[j]],
                                        add=True)
                    jn = jnp.minimum(j + _NBUF, _KH - 1)
                    pltpu.async_copy(h_hbm.at[src_v.at[jn]], rows[b], sems[b])
                return carry

            lax.fori_loop(0, _KH // _NBUF, body, 0)

            for b in range(_NBUF):
                pltpu.make_async_copy(h_hbm.at[src_v.at[_KH - 1]], rows[b],
                                      sems[b]).wait()

        plsc.subcore_barrier()

        pltpu.sync_copy(sh_agg.at[pl.ds(r0, _RPT), :],
                        agg_out.at[cid, pl.ds(r0, _RPT), :])

        if with_deg:
            @pl.when(sid == 0)
            def _():
                pltpu.sync_copy(sh_deg, deg_out.at[cid])

    return agg


def _make_tc_layer(relu):
    def body(h_ref, agg_ref, deg_ref, ws_ref, wn_ref, b_ref, o_ref):
        d = deg_ref[0] + deg_ref[1]
        inv = 1.0 / jnp.maximum(d, 1.0)
        hn = (agg_ref[0] + agg_ref[1]) * inv
        acc = jnp.dot(h_ref[...], ws_ref[...], preferred_element_type=jnp.float32)
        acc = acc + jnp.dot(hn, wn_ref[...], preferred_element_type=jnp.float32)
        acc = acc + b_ref[...]
        if relu:
            acc = jnp.maximum(acc, 0.0)
        o_ref[...] = acc

    return pl.pallas_call(
        body,
        grid=(_GRID,),
        in_specs=[
            pl.BlockSpec((_BN, _D), lambda i: (i, 0)),
            pl.BlockSpec((2, _BN, _D), lambda i: (0, i, 0)),
            pl.BlockSpec((2, _BN, 1), lambda i: (0, i, 0)),
            pl.BlockSpec((_D, _D), lambda i: (0, 0)),
            pl.BlockSpec((_D, _D), lambda i: (0, 0)),
            pl.BlockSpec((1, _D), lambda i: (0, 0)),
        ],
        out_specs=pl.BlockSpec((_BN, _D), lambda i: (i, 0)),
        out_shape=jax.ShapeDtypeStruct((_NP, _D), jnp.float32),
    )


def kernel(x, edge_index, W_self0, W_neigh0, b0, W_self1, W_neigh1, b1):
    src = edge_index[0]
    dst = edge_index[1]
    extra = _EPAD - _E
    pad_src = jnp.arange(extra, dtype=jnp.int32) % _N
    pad_dst = _N + (jnp.arange(extra, dtype=jnp.int32) % (_NP - _N))
    src3 = (jnp.concatenate([src, pad_src])
            .reshape(_K * _C, _NW).T.reshape(_NW, _K, _C))
    dst3 = (jnp.concatenate([dst, pad_dst])
            .reshape(_K * _C, _NW).T.reshape(_NW, _K, _C))
    x_pad = jnp.pad(x, ((0, _NP - _N), (0, 0)))
    z2d = jnp.zeros((_RPT, _D), jnp.float32)
    z1d = jnp.zeros((_RPT,), jnp.float32)
    b0r = b0.reshape(1, _D)
    b1r = b1.reshape(1, _D)

    agg0, deg = _make_sc_aggregate(True)(x_pad, src3, dst3, z2d, z1d)
    deg3 = deg.reshape(2, _NP, 1)
    h1 = _make_tc_layer(True)(x_pad, agg0, deg3, W_self0, W_neigh0, b0r)
    (agg1,) = _make_sc_aggregate(False)(h1, src3, dst3, z2d, z1d)
    out = _make_tc_layer(False)(h1, agg1, deg3, W_self1, W_neigh1, b1r)
    return out[:_N]
